# baseline (speedup 1.0000x reference)
import functools

import jax
import jax.numpy as jnp
from jax import lax
from jax.experimental import pallas as pl
from jax.experimental.pallas import tpu as pltpu
from jax.experimental.pallas import tpu_sc as plsc

NUM_ENTITIES = 1000000
NUM_RELATIONS = 1000
D = 64
TIMESTEPS = 1000
B = 16384

NC, NS, L = 2, 16, 16
NW = NC * NS
CH = 128
NCH = (B // NW) // CH
IDX_ROWS = B // CH

_sc_mesh = plsc.VectorSubcoreMesh(core_axis_name="c", subcore_axis_name="s")


@functools.partial(
    pl.kernel,
    out_type=[
        jax.ShapeDtypeStruct((IDX_ROWS, CH, D), jnp.float32),
        jax.ShapeDtypeStruct((IDX_ROWS, CH, D), jnp.float32),
        jax.ShapeDtypeStruct((IDX_ROWS, CH, D), jnp.float32),
        jax.ShapeDtypeStruct((IDX_ROWS, CH), jnp.float32),
    ],
    mesh=_sc_mesh,
    compiler_params=pltpu.CompilerParams(
        needs_layout_passes=False, use_tc_tiling_on_sc=False),
    scratch_types=[
        pltpu.VMEM((NCH, CH), jnp.int32),
        pltpu.VMEM((NCH, CH), jnp.int32),
        pltpu.VMEM((NCH, CH), jnp.int32),
        pltpu.VMEM((NCH, CH), jnp.int32),
        pltpu.VMEM((NCH, CH, D), jnp.float32),
        pltpu.VMEM((NCH, CH, D), jnp.float32),
        pltpu.VMEM((NCH, CH, D), jnp.float32),
        pltpu.VMEM((1008,), jnp.float32),
        pltpu.VMEM((NCH, CH), jnp.float32),
        pltpu.SemaphoreType.DMA,
        pltpu.SemaphoreType.DMA,
        pltpu.SemaphoreType.DMA,
    ],
)
def _sc_gather(ent_hbm, rel_hbm, tstab_hbm, hidx_hbm, tidx_hbm, ridx_hbm,
               tsidx_hbm, h_out, t_out, r_out, temb_out,
               hidx_v, tidx_v, ridx_v, tsidx_v, h_v, t_v, r_v, tstab_v,
               temb_v, sem_h, sem_t, sem_r):
    wid = lax.axis_index("s") * NC + lax.axis_index("c")
    base = wid * NCH
    pltpu.sync_copy(hidx_hbm.at[pl.ds(base, NCH)], hidx_v)
    pltpu.sync_copy(tidx_hbm.at[pl.ds(base, NCH)], tidx_v)
    pltpu.sync_copy(ridx_hbm.at[pl.ds(base, NCH)], ridx_v)
    copies = []
    for j in range(NCH):
        copies.append(pltpu.async_copy(ent_hbm.at[hidx_v.at[j]], h_v.at[j], sem_h))
        copies.append(pltpu.async_copy(ent_hbm.at[tidx_v.at[j]], t_v.at[j], sem_t))
        copies.append(pltpu.async_copy(rel_hbm.at[ridx_v.at[j]], r_v.at[j], sem_r))
    pltpu.sync_copy(tsidx_hbm.at[pl.ds(base, NCH)], tsidx_v)
    pltpu.sync_copy(tstab_hbm, tstab_v)
    for j in range(NCH):
        for c in range(CH // L):
            idx16 = tsidx_v[j, pl.ds(c * L, L)]
            temb_v[j, pl.ds(c * L, L)] = plsc.load_gather(tstab_v, [idx16])
    pltpu.sync_copy(temb_v, temb_out.at[pl.ds(base, NCH)])
    for cp in copies:
        cp.wait()
    pltpu.sync_copy(h_v, h_out.at[pl.ds(base, NCH)])
    pltpu.sync_copy(t_v, t_out.at[pl.ds(base, NCH)])
    pltpu.sync_copy(r_v, r_out.at[pl.ds(base, NCH)])


BLK = 2048
GRID = B // BLK


def _tc_body(ts_ref, temb_ref, h_ref, t_ref, r_ref, nz_ref,
             w1rt_ref, w1l_ref, b1_ref, w2t_ref, b2_ref,
             score_ref, loss_ref):
    alpha = 1.0 - ts_ref[...].astype(jnp.float32) * (1.0 / TIMESTEPS)
    r = r_ref[...]
    nz = nz_ref[...]
    rt = alpha * r + (1.0 - alpha) * nz
    x1 = jnp.dot(rt, w1rt_ref[...], preferred_element_type=jnp.float32)
    x1 = x1 + temb_ref[...] * w1l_ref[...] + b1_ref[...]
    hdn = jnp.maximum(x1, 0.0)
    pred = jnp.dot(hdn, w2t_ref[...], preferred_element_type=jnp.float32)
    pred = pred + b2_ref[...]
    dn = pred - nz
    part = jnp.sum(dn * dn) * (1.0 / (B * D))

    @pl.when(pl.program_id(0) == 0)
    def _():
        loss_ref[...] = jnp.zeros_like(loss_ref)

    loss_ref[...] = loss_ref[...] + jnp.full((1, 128), part, jnp.float32)
    sc = h_ref[...] + rt - t_ref[...]
    s2 = jnp.sum(sc * sc, axis=1)
    score_ref[...] = (-jnp.sqrt(s2)).reshape(1, 1, BLK)


def kernel(h_idx, r_idx, t_idx, time_step, entity_emb, relation_emb,
           timestep_emb, W1, b1, W2, b2):
    hidx = h_idx.astype(jnp.int32).reshape(IDX_ROWS, CH)
    tidx = t_idx.astype(jnp.int32).reshape(IDX_ROWS, CH)
    ridx = r_idx.astype(jnp.int32).reshape(IDX_ROWS, CH)
    tsidx = time_step.astype(jnp.int32).reshape(IDX_ROWS, CH)
    tstab = jnp.concatenate(
        [timestep_emb[:, 0], jnp.zeros((1008 - (TIMESTEPS + 1),), jnp.float32)])
    h_g, t_g, r_g, temb_g = _sc_gather(
        entity_emb, relation_emb, tstab, hidx, tidx, ridx, tsidx)
    h2 = h_g.reshape(B, D)
    t2 = t_g.reshape(B, D)
    r2 = r_g.reshape(B, D)
    temb2 = temb_g.reshape(B, 1)

    noise = jax.random.normal(jax.random.key(42), (B, D), dtype=jnp.float32)
    W1rt = W1[:, :D].T
    w1l = W1[:, D].reshape(1, D)
    b1r = b1.reshape(1, D)
    W2T = W2.T
    b2r = b2.reshape(1, D)

    score3, lossp = pl.pallas_call(
        _tc_body,
        grid=(GRID,),
        in_specs=[
            pl.BlockSpec((BLK, 1), lambda i: (i, 0)),
            pl.BlockSpec((BLK, 1), lambda i: (i, 0)),
            pl.BlockSpec((BLK, D), lambda i: (i, 0)),
            pl.BlockSpec((BLK, D), lambda i: (i, 0)),
            pl.BlockSpec((BLK, D), lambda i: (i, 0)),
            pl.BlockSpec((BLK, D), lambda i: (i, 0)),
            pl.BlockSpec((D, D), lambda i: (0, 0)),
            pl.BlockSpec((1, D), lambda i: (0, 0)),
            pl.BlockSpec((1, D), lambda i: (0, 0)),
            pl.BlockSpec((D, D), lambda i: (0, 0)),
            pl.BlockSpec((1, D), lambda i: (0, 0)),
        ],
        out_specs=[
            pl.BlockSpec((1, 1, BLK), lambda i: (i, 0, 0)),
            pl.BlockSpec((1, 128), lambda i: (0, 0)),
        ],
        out_shape=[
            jax.ShapeDtypeStruct((GRID, 1, BLK), jnp.float32),
            jax.ShapeDtypeStruct((1, 128), jnp.float32),
        ],
    )(time_step, temb2, h2, t2, r2, noise, W1rt, w1l, b1r, W2T, b2r)
    loss = lossp[0, 0]
    score = score3.reshape(B)
    return (loss, score)


# speedup vs baseline: 1.0537x; 1.0537x over previous
import functools

import jax
import jax.numpy as jnp
from jax import lax
from jax.experimental import pallas as pl
from jax.experimental.pallas import tpu as pltpu
from jax.experimental.pallas import tpu_sc as plsc

NUM_ENTITIES = 1000000
NUM_RELATIONS = 1000
D = 64
TIMESTEPS = 1000
B = 16384

NC, NS, L = 2, 16, 16
NW = NC * NS
CH = 128
NCH = (B // NW) // CH
IDX_ROWS = B // CH
ROWS_W = B // NW
BP = B // 2


@functools.cache
def _make_sc_gather():
    mesh = plsc.VectorSubcoreMesh(
        core_axis_name="c", subcore_axis_name="s",
        num_cores=NC, num_subcores=NS)
    return pl.kernel(
        _sc_gather_body,
        out_type=[
            jax.ShapeDtypeStruct((IDX_ROWS, CH, D), jnp.float32),
            jax.ShapeDtypeStruct((IDX_ROWS, CH, D), jnp.float32),
            jax.ShapeDtypeStruct((IDX_ROWS, CH), jnp.float32),
        ],
        mesh=mesh,
        compiler_params=pltpu.CompilerParams(
            needs_layout_passes=False, use_tc_tiling_on_sc=False),
        scratch_types=[
            pltpu.VMEM((NCH, CH), jnp.int32),
            pltpu.VMEM((NCH, CH), jnp.int32),
            pltpu.VMEM((NCH, CH), jnp.int32),
            pltpu.VMEM((NCH, CH), jnp.int32),
            pltpu.VMEM((NCH, CH, D), jnp.float32),
            pltpu.VMEM((NCH, CH, D), jnp.float32),
            pltpu.VMEM((NCH, CH, D), jnp.float32),
            pltpu.VMEM((1008,), jnp.float32),
            pltpu.VMEM((NCH, CH), jnp.float32),
            pltpu.SemaphoreType.DMA,
            pltpu.SemaphoreType.DMA,
            pltpu.SemaphoreType.DMA,
        ],
    )


def _sc_gather_body(ent_hbm, rel_hbm, tstab_hbm, hidx_hbm, tidx_hbm, ridx_hbm,
                    tsidx_hbm, hmt_out, r_out, temb_out,
                    hidx_v, tidx_v, ridx_v, tsidx_v, h_v, t_v, r_v, tstab_v,
                    temb_v, sem_h, sem_t, sem_r):
    wid = lax.axis_index("s") * NC + lax.axis_index("c")
    base = wid * NCH
    pltpu.sync_copy(hidx_hbm.at[pl.ds(base, NCH)], hidx_v)
    pltpu.sync_copy(tidx_hbm.at[pl.ds(base, NCH)], tidx_v)
    pltpu.sync_copy(ridx_hbm.at[pl.ds(base, NCH)], ridx_v)
    copies = []
    for j in range(NCH):
        copies.append(pltpu.async_copy(ent_hbm.at[hidx_v.at[j]], h_v.at[j],
                                       sem_h))
        copies.append(pltpu.async_copy(ent_hbm.at[tidx_v.at[j]], t_v.at[j],
                                       sem_t))
        copies.append(pltpu.async_copy(rel_hbm.at[ridx_v.at[j]], r_v.at[j],
                                       sem_r))
    pltpu.sync_copy(tsidx_hbm.at[pl.ds(base, NCH)], tsidx_v)
    pltpu.sync_copy(tstab_hbm, tstab_v)
    for j in range(NCH):
        for c in range(CH // L):
            idx16 = tsidx_v[j, pl.ds(c * L, L)]
            temb_v[j, pl.ds(c * L, L)] = plsc.load_gather(tstab_v, [idx16])
    pltpu.sync_copy(temb_v, temb_out.at[pl.ds(base, NCH)])
    for cp in copies:
        cp.wait()

    def _sub(p, carry):
        for j in range(NCH):
            for c in range(D // L):
                sl = pl.ds(c * L, L)
                h_v[j, p, sl] = h_v[j, p, sl] - t_v[j, p, sl]
        return carry

    lax.fori_loop(0, CH, _sub, 0)

    pltpu.sync_copy(h_v, hmt_out.at[pl.ds(base, NCH)])
    pltpu.sync_copy(r_v, r_out.at[pl.ds(base, NCH)])


PBLK = 1024
GRID = BP // PBLK


def _pack2(v):
    left = jnp.broadcast_to(v[:, 0:1], (PBLK, D))
    right = jnp.broadcast_to(v[:, 1:2], (PBLK, D))
    return jnp.concatenate([left, right], axis=1)


def _tc_body(ts_ref, temb_ref, hmt_ref, r_ref, nz_ref,
             w1bd_ref, w1l2_ref, b1t_ref, w2bd_ref, b2t_ref, ones_ref,
             score_ref, loss_ref):
    alpha = _pack2(1.0 - ts_ref[...].astype(jnp.float32) * (1.0 / TIMESTEPS))
    tex = _pack2(temb_ref[...])
    r = r_ref[...]
    nz = nz_ref[...]
    rt = alpha * r + (1.0 - alpha) * nz
    x1 = jnp.dot(rt, w1bd_ref[...], preferred_element_type=jnp.float32)
    x1 = x1 + tex * w1l2_ref[...] + b1t_ref[...]
    hdn = jnp.maximum(x1, 0.0)
    pred = jnp.dot(hdn, w2bd_ref[...], preferred_element_type=jnp.float32)
    pred = pred + b2t_ref[...]
    dn = pred - nz
    part = jnp.sum(dn * dn) * (1.0 / (B * D))

    @pl.when(pl.program_id(0) == 0)
    def _():
        loss_ref[...] = jnp.zeros_like(loss_ref)

    loss_ref[...] = loss_ref[...] + jnp.full((1, 128), part, jnp.float32)
    s = hmt_ref[...] + rt
    s2 = jnp.dot(s * s, ones_ref[...], preferred_element_type=jnp.float32)
    score_ref[...] = -jnp.sqrt(s2)


def kernel(h_idx, r_idx, t_idx, time_step, entity_emb, relation_emb,
           timestep_emb, W1, b1, W2, b2):
    hidx = h_idx.astype(jnp.int32).reshape(IDX_ROWS, CH)
    tidx = t_idx.astype(jnp.int32).reshape(IDX_ROWS, CH)
    ridx = r_idx.astype(jnp.int32).reshape(IDX_ROWS, CH)
    tsidx = time_step.astype(jnp.int32).reshape(IDX_ROWS, CH)
    tstab = jnp.concatenate(
        [timestep_emb[:, 0], jnp.zeros((1008 - (TIMESTEPS + 1),), jnp.float32)])
    hmt_g, r_g, temb_g = _make_sc_gather()(
        entity_emb, relation_emb, tstab, hidx, tidx, ridx, tsidx)

    noise = jax.random.normal(jax.random.key(42), (BP, 2 * D),
                              dtype=jnp.float32)
    W1rt = W1[:, :D].T
    zero = jnp.zeros((D, D), jnp.float32)
    W1bd = jnp.block([[W1rt, zero], [zero, W1rt]])
    W2T = W2.T
    W2bd = jnp.block([[W2T, zero], [zero, W2T]])
    w1l2 = jnp.tile(W1[:, D].reshape(1, D), (1, 2))
    b1t = jnp.tile(b1.reshape(1, D), (1, 2))
    b2t = jnp.tile(b2.reshape(1, D), (1, 2))
    onecol = jnp.ones((D, 1), jnp.float32)
    zcol = jnp.zeros((D, 1), jnp.float32)
    onesbd = jnp.block([[onecol, zcol], [zcol, onecol]])
    ts2 = time_step.reshape(BP, 2)
    temb2 = temb_g.reshape(BP, 2)
    hmt2 = hmt_g.reshape(BP, 2 * D)
    r2 = r_g.reshape(BP, 2 * D)

    score2, lossp = pl.pallas_call(
        _tc_body,
        grid=(GRID,),
        in_specs=[
            pl.BlockSpec((PBLK, 2), lambda i: (i, 0)),
            pl.BlockSpec((PBLK, 2), lambda i: (i, 0)),
            pl.BlockSpec((PBLK, 2 * D), lambda i: (i, 0)),
            pl.BlockSpec((PBLK, 2 * D), lambda i: (i, 0)),
            pl.BlockSpec((PBLK, 2 * D), lambda i: (i, 0)),
            pl.BlockSpec((2 * D, 2 * D), lambda i: (0, 0)),
            pl.BlockSpec((1, 2 * D), lambda i: (0, 0)),
            pl.BlockSpec((1, 2 * D), lambda i: (0, 0)),
            pl.BlockSpec((2 * D, 2 * D), lambda i: (0, 0)),
            pl.BlockSpec((1, 2 * D), lambda i: (0, 0)),
            pl.BlockSpec((2 * D, 2), lambda i: (0, 0)),
        ],
        out_specs=[
            pl.BlockSpec((PBLK, 2), lambda i: (i, 0)),
            pl.BlockSpec((1, 128), lambda i: (0, 0)),
        ],
        out_shape=[
            jax.ShapeDtypeStruct((BP, 2), jnp.float32),
            jax.ShapeDtypeStruct((1, 128), jnp.float32),
        ],
    )(ts2, temb2, hmt2, r2, noise, W1bd, w1l2, b1t, W2bd, b2t, onesbd)
    loss = lossp[0, 0]
    score = score2.reshape(B)
    return (loss, score)


# speedup vs baseline: 1.7098x; 1.6226x over previous
import functools

import jax
import jax.numpy as jnp
from jax import lax
from jax.experimental import pallas as pl
from jax.experimental.pallas import tpu as pltpu
from jax.experimental.pallas import tpu_sc as plsc

NUM_ENTITIES = 1000000
NUM_RELATIONS = 1000
D = 64
TIMESTEPS = 1000
B = 16384

NC, NS, L = 2, 16, 16
NW = NC * NS
ROWS_W = B // NW
PACK_W = ROWS_W // 2
BP = B // 2


@functools.cache
def _make_sc_gather():
    mesh = plsc.VectorSubcoreMesh(
        core_axis_name="c", subcore_axis_name="s",
        num_cores=NC, num_subcores=NS)
    return pl.kernel(
        _sc_gather_body,
        out_type=[
            jax.ShapeDtypeStruct((BP, 2 * D), jnp.float32),
            jax.ShapeDtypeStruct((BP, 2 * D), jnp.float32),
            jax.ShapeDtypeStruct((B // 128, 128), jnp.float32),
        ],
        mesh=mesh,
        compiler_params=pltpu.CompilerParams(
            needs_layout_passes=False, use_tc_tiling_on_sc=True),
        scratch_types=[
            pltpu.VMEM((ROWS_W,), jnp.int32),
            pltpu.VMEM((ROWS_W,), jnp.int32),
            pltpu.VMEM((ROWS_W,), jnp.int32),
            pltpu.VMEM((ROWS_W,), jnp.int32),
            pltpu.VMEM((PACK_W, 2 * D), jnp.float32),
            pltpu.VMEM((PACK_W, 2 * D), jnp.float32),
            pltpu.VMEM((PACK_W, 2 * D), jnp.float32),
            pltpu.VMEM((1008,), jnp.float32),
            pltpu.VMEM((4, 128), jnp.float32),
            pltpu.SemaphoreType.DMA,
            pltpu.SemaphoreType.DMA,
            pltpu.SemaphoreType.DMA,
        ],
    )


def _sc_gather_body(ent_hbm, rel_hbm, tstab_hbm, hidx_hbm, tidx_hbm, ridx_hbm,
                    tsidx_hbm, hmt_out, r_out, temb_out,
                    hidx_v, tidx_v, ridx_v, tsidx_v, h_v, t_v, r_v, tstab_v,
                    temb_v, sem_h, sem_t, sem_r):
    wid = lax.axis_index("s") * NC + lax.axis_index("c")
    rbase = wid * ROWS_W
    pbase = wid * PACK_W
    pltpu.sync_copy(hidx_hbm.at[pl.ds(rbase, ROWS_W)], hidx_v)
    pltpu.sync_copy(tidx_hbm.at[pl.ds(rbase, ROWS_W)], tidx_v)
    pltpu.sync_copy(ridx_hbm.at[pl.ds(rbase, ROWS_W)], ridx_v)
    pltpu.sync_copy(tsidx_hbm.at[pl.ds(rbase, ROWS_W)], tsidx_v)
    pltpu.sync_copy(tstab_hbm, tstab_v)

    def _fire(c, carry):
        hv16 = hidx_v[pl.ds(c * L, L)]
        tv16 = tidx_v[pl.ds(c * L, L)]
        rv16 = ridx_v[pl.ds(c * L, L)]
        for k in range(L):
            prow = c * (L // 2) + k // 2
            sl = pl.ds((k % 2) * D, D)
            pltpu.async_copy(ent_hbm.at[hv16[k]], h_v.at[prow, sl], sem_h)
            pltpu.async_copy(ent_hbm.at[tv16[k]], t_v.at[prow, sl], sem_t)
            pltpu.async_copy(rel_hbm.at[rv16[k]], r_v.at[prow, sl], sem_r)
        return carry

    lax.fori_loop(0, ROWS_W // L, _fire, 0)

    for c in range(ROWS_W // L):
        idx16 = tsidx_v[pl.ds(c * L, L)]
        temb_v[c // 8, pl.ds((c % 8) * L, L)] = plsc.load_gather(
            tstab_v, [idx16])

    pltpu.make_async_copy(hmt_out.at[pl.ds(0, PACK_W)], h_v, sem_h).wait()
    pltpu.make_async_copy(hmt_out.at[pl.ds(0, PACK_W)], t_v, sem_t).wait()

    def _sub(p, carry):
        for c in range(2 * D // L):
            sl = pl.ds(c * L, L)
            h_v[p, sl] = h_v[p, sl] - t_v[p, sl]
        return carry

    lax.fori_loop(0, PACK_W, _sub, 0)

    pltpu.make_async_copy(r_out.at[pl.ds(0, PACK_W)], r_v, sem_r).wait()

    pltpu.sync_copy(h_v, hmt_out.at[pl.ds(pbase, PACK_W)])
    pltpu.sync_copy(r_v, r_out.at[pl.ds(pbase, PACK_W)])
    pltpu.sync_copy(temb_v, temb_out.at[pl.ds(wid * 4, 4)])


PBLK = 1024
GRID = BP // PBLK


def _pack2(v):
    left = jnp.broadcast_to(v[:, 0:1], (PBLK, D))
    right = jnp.broadcast_to(v[:, 1:2], (PBLK, D))
    return jnp.concatenate([left, right], axis=1)


def _tc_body(ts_ref, temb_ref, hmt_ref, r_ref, nz_ref,
             w1bd_ref, w1l2_ref, b1t_ref, w2bd_ref, b2t_ref, ones_ref,
             score_ref, loss_ref):
    alpha = _pack2(1.0 - ts_ref[...].astype(jnp.float32) * (1.0 / TIMESTEPS))
    tex = _pack2(temb_ref[...])
    r = r_ref[...]
    nz = nz_ref[...]
    rt = alpha * r + (1.0 - alpha) * nz
    x1 = jnp.dot(rt, w1bd_ref[...], preferred_element_type=jnp.float32)
    x1 = x1 + tex * w1l2_ref[...] + b1t_ref[...]
    hdn = jnp.maximum(x1, 0.0)
    pred = jnp.dot(hdn, w2bd_ref[...], preferred_element_type=jnp.float32)
    pred = pred + b2t_ref[...]
    dn = pred - nz
    part = jnp.sum(dn * dn) * (1.0 / (B * D))

    @pl.when(pl.program_id(0) == 0)
    def _():
        loss_ref[...] = jnp.zeros_like(loss_ref)

    loss_ref[...] = loss_ref[...] + jnp.full((1, 128), part, jnp.float32)
    s = hmt_ref[...] + rt
    s2 = jnp.dot(s * s, ones_ref[...], preferred_element_type=jnp.float32)
    score_ref[...] = -jnp.sqrt(s2)


def kernel(h_idx, r_idx, t_idx, time_step, entity_emb, relation_emb,
           timestep_emb, W1, b1, W2, b2):
    hidx = h_idx.astype(jnp.int32)
    tidx = t_idx.astype(jnp.int32)
    ridx = r_idx.astype(jnp.int32)
    tsidx = time_step.astype(jnp.int32).reshape(B)
    tstab = jnp.concatenate(
        [timestep_emb[:, 0], jnp.zeros((1008 - (TIMESTEPS + 1),), jnp.float32)])
    hmt_g, r_g, temb_g = _make_sc_gather()(
        entity_emb, relation_emb, tstab, hidx, tidx, ridx, tsidx)

    noise = jax.random.normal(jax.random.key(42), (BP, 2 * D),
                              dtype=jnp.float32)
    W1rt = W1[:, :D].T
    zero = jnp.zeros((D, D), jnp.float32)
    W1bd = jnp.block([[W1rt, zero], [zero, W1rt]])
    W2T = W2.T
    W2bd = jnp.block([[W2T, zero], [zero, W2T]])
    w1l2 = jnp.tile(W1[:, D].reshape(1, D), (1, 2))
    b1t = jnp.tile(b1.reshape(1, D), (1, 2))
    b2t = jnp.tile(b2.reshape(1, D), (1, 2))
    onecol = jnp.ones((D, 1), jnp.float32)
    zcol = jnp.zeros((D, 1), jnp.float32)
    onesbd = jnp.block([[onecol, zcol], [zcol, onecol]])
    ts2 = time_step.reshape(BP, 2)
    temb2 = temb_g.reshape(BP, 2)

    score2, lossp = pl.pallas_call(
        _tc_body,
        grid=(GRID,),
        in_specs=[
            pl.BlockSpec((PBLK, 2), lambda i: (i, 0)),
            pl.BlockSpec((PBLK, 2), lambda i: (i, 0)),
            pl.BlockSpec((PBLK, 2 * D), lambda i: (i, 0)),
            pl.BlockSpec((PBLK, 2 * D), lambda i: (i, 0)),
            pl.BlockSpec((PBLK, 2 * D), lambda i: (i, 0)),
            pl.BlockSpec((2 * D, 2 * D), lambda i: (0, 0)),
            pl.BlockSpec((1, 2 * D), lambda i: (0, 0)),
            pl.BlockSpec((1, 2 * D), lambda i: (0, 0)),
            pl.BlockSpec((2 * D, 2 * D), lambda i: (0, 0)),
            pl.BlockSpec((1, 2 * D), lambda i: (0, 0)),
            pl.BlockSpec((2 * D, 2), lambda i: (0, 0)),
        ],
        out_specs=[
            pl.BlockSpec((PBLK, 2), lambda i: (i, 0)),
            pl.BlockSpec((1, 128), lambda i: (0, 0)),
        ],
        out_shape=[
            jax.ShapeDtypeStruct((BP, 2), jnp.float32),
            jax.ShapeDtypeStruct((1, 128), jnp.float32),
        ],
    )(ts2, temb2, hmt_g, r_g, noise, W1bd, w1l2, b1t, W2bd, b2t, onesbd)
    loss = lossp[0, 0]
    score = score2.reshape(B)
    return (loss, score)
